# baseline (speedup 1.0000x reference)
import functools
import math

import jax
import jax.numpy as jnp
from jax import lax
from jax.experimental import pallas as pl
from jax.experimental.pallas import tpu as pltpu
from jax.experimental.pallas import tpu_sc as plsc

B = 64
S = 8192
OUTPUT_DIM = 64
NUM_BINS = 32
MAX_DISTANCE = 1e7
HALF = OUTPUT_DIM // 2

_INV_SQRT2 = 0.7071067811865476
_RB = 8


def _encoder_kernel(d_ref, edges_ref, w1_ref, b1_ref, wf_ref, bias_ref, out_ref):
    r = pl.program_id(1)
    d = d_ref[pl.ds(r, 1), :]
    cmp_t = (d > edges_ref[...]).astype(jnp.float32)
    ld = jnp.log1p(d * 1e-3)
    pre = ld * w1_ref[...] + b1_ref[...]
    h_t = 0.5 * pre * (1.0 + jax.lax.erf(pre * _INV_SQRT2))
    xt = jnp.concatenate([cmp_t, h_t], axis=0)
    out_ref[...] = (
        jax.lax.dot_general(
            xt, wf_ref[...],
            dimension_numbers=(((0,), (0,)), ((), ())),
            preferred_element_type=jnp.float32,
        )
        + bias_ref[...]
    )


def kernel(distances, emb, W1, b1, W2, b2, Wc, bc):
    N = B * S

    edges = jnp.logspace(3.0, math.log10(MAX_DISTANCE), NUM_BINS,
                         dtype=jnp.float32)
    Temb = emb @ Wc[:OUTPUT_DIM]
    dT = Temb[1:] - Temb[:-1]
    Wh = W2 @ Wc[OUTPUT_DIM:]
    Wf = jnp.concatenate([dT, Wh], axis=0)
    bias = (Temb[0] + b2 @ Wc[OUTPUT_DIM:] + bc).reshape(1, OUTPUT_DIM)

    edges_col = edges.reshape(NUM_BINS, 1)
    w1_col = W1.reshape(HALF, 1)
    b1_col = b1.reshape(HALF, 1)

    grid = (B // _RB, _RB)
    out = pl.pallas_call(
        _encoder_kernel,
        grid=grid,
        in_specs=[
            pl.BlockSpec((_RB, S), lambda i, j: (i, 0)),
            pl.BlockSpec((NUM_BINS, 1), lambda i, j: (0, 0)),
            pl.BlockSpec((HALF, 1), lambda i, j: (0, 0)),
            pl.BlockSpec((HALF, 1), lambda i, j: (0, 0)),
            pl.BlockSpec((OUTPUT_DIM, OUTPUT_DIM), lambda i, j: (0, 0)),
            pl.BlockSpec((1, OUTPUT_DIM), lambda i, j: (0, 0)),
        ],
        out_specs=pl.BlockSpec((S, OUTPUT_DIM), lambda i, j: (i * _RB + j, 0)),
        out_shape=jax.ShapeDtypeStruct((N, OUTPUT_DIM), jnp.float32),
    )(distances, edges_col, w1_col, b1_col, Wf, bias)

    return _relayout_sc(out)


def _relayout_sc(dense):
    info = plsc.get_sparse_core_info()
    nw = info.num_cores * info.num_subcores
    rows_per_w = (B * S) // nw
    ch = 512
    n_ch = rows_per_w // ch
    mesh = plsc.VectorSubcoreMesh(core_axis_name="c", subcore_axis_name="s")

    @functools.partial(
        pl.kernel, mesh=mesh,
        out_type=jax.ShapeDtypeStruct((B, S, OUTPUT_DIM), jnp.float32),
        scratch_types=[
            pltpu.VMEM((2, ch, OUTPUT_DIM), jnp.float32),
        ],
    )
    def _k(dense_hbm, out_hbm, buf):
        wid = lax.axis_index("s") * info.num_cores + lax.axis_index("c")
        base = wid * rows_per_w
        for ci in range(n_ch):
            row0 = base + ci * ch
            b = row0 // S
            s0 = lax.rem(row0, S)
            slot = ci % 2
            pltpu.sync_copy(dense_hbm.at[pl.ds(row0, ch)], buf.at[slot])
            pltpu.sync_copy(buf.at[slot], out_hbm.at[b, pl.ds(s0, ch)])

    return _k(dense)


# speedup vs baseline: 1.0197x; 1.0197x over previous
import functools
import math

import jax
import jax.numpy as jnp
from jax import lax
from jax.experimental import pallas as pl
from jax.experimental.pallas import tpu as pltpu
from jax.experimental.pallas import tpu_sc as plsc

B = 64
S = 8192
OUTPUT_DIM = 64
NUM_BINS = 32
MAX_DISTANCE = 1e7
HALF = OUTPUT_DIM // 2

_INV_SQRT2 = 0.7071067811865476
_RB = 8


def _encoder_kernel(d_ref, edges_ref, w1_ref, b1_ref, wf_ref, bias_ref, out_ref):
    r = pl.program_id(1)
    d = d_ref[pl.ds(r, 1), :]
    cmp_t = (d > edges_ref[...]).astype(jnp.float32)
    ld = jnp.log1p(d * 1e-3)
    pre = ld * w1_ref[...] + b1_ref[...]
    h_t = 0.5 * pre * (1.0 + jax.lax.erf(pre * _INV_SQRT2))
    xt = jnp.concatenate([cmp_t, h_t], axis=0)
    out_ref[...] = (
        jax.lax.dot_general(
            xt, wf_ref[...],
            dimension_numbers=(((0,), (0,)), ((), ())),
            preferred_element_type=jnp.float32,
        )
        + bias_ref[...]
    )


def kernel(distances, emb, W1, b1, W2, b2, Wc, bc):
    N = B * S

    edges = jnp.logspace(3.0, math.log10(MAX_DISTANCE), NUM_BINS,
                         dtype=jnp.float32)
    Temb = emb @ Wc[:OUTPUT_DIM]
    dT = Temb[1:] - Temb[:-1]
    Wh = W2 @ Wc[OUTPUT_DIM:]
    Wf = jnp.concatenate([dT, Wh], axis=0)
    bias = (Temb[0] + b2 @ Wc[OUTPUT_DIM:] + bc).reshape(1, OUTPUT_DIM)

    edges_col = edges.reshape(NUM_BINS, 1)
    w1_col = W1.reshape(HALF, 1)
    b1_col = b1.reshape(HALF, 1)

    grid = (B // _RB, _RB)
    out = pl.pallas_call(
        _encoder_kernel,
        grid=grid,
        in_specs=[
            pl.BlockSpec((_RB, S), lambda i, j: (i, 0)),
            pl.BlockSpec((NUM_BINS, 1), lambda i, j: (0, 0)),
            pl.BlockSpec((HALF, 1), lambda i, j: (0, 0)),
            pl.BlockSpec((HALF, 1), lambda i, j: (0, 0)),
            pl.BlockSpec((OUTPUT_DIM, OUTPUT_DIM), lambda i, j: (0, 0)),
            pl.BlockSpec((1, OUTPUT_DIM), lambda i, j: (0, 0)),
        ],
        out_specs=pl.BlockSpec((S, OUTPUT_DIM), lambda i, j: (i * _RB + j, 0)),
        out_shape=jax.ShapeDtypeStruct((N, OUTPUT_DIM), jnp.float32),
    )(distances, edges_col, w1_col, b1_col, Wf, bias)

    return _relayout_sc(out)


def _relayout_sc(dense):
    info = plsc.get_sparse_core_info()
    nw = info.num_cores * info.num_subcores
    rows_per_w = (B * S) // nw
    ch = 256
    n_ch = rows_per_w // ch
    mesh = plsc.VectorSubcoreMesh(core_axis_name="c", subcore_axis_name="s")

    @functools.partial(
        pl.kernel, mesh=mesh,
        out_type=jax.ShapeDtypeStruct((B, S, OUTPUT_DIM), jnp.float32),
        scratch_types=[
            pltpu.VMEM((4, ch, OUTPUT_DIM), jnp.float32),
            pltpu.SemaphoreType.DMA((4,)),
            pltpu.SemaphoreType.DMA((4,)),
        ],
    )
    def _k(dense_hbm, out_hbm, buf, sin, sout):
        wid = lax.axis_index("s") * info.num_cores + lax.axis_index("c")
        base = wid * rows_per_w

        def in_copy(ci):
            row0 = base + ci * ch
            return pltpu.make_async_copy(
                dense_hbm.at[pl.ds(row0, ch)], buf.at[ci % 4],
                sin.at[ci % 4])

        def out_copy(ci):
            row0 = base + ci * ch
            return pltpu.make_async_copy(
                buf.at[ci % 4], out_hbm.at[row0 // S, pl.ds(lax.rem(row0, S), ch)],
                sout.at[ci % 4])

        in_copy(0).start()
        in_copy(1).start()
        for ci in range(n_ch):
            in_copy(ci).wait()
            out_copy(ci).start()
            if ci >= 2:
                out_copy(ci - 2).wait()
            if ci + 2 < n_ch:
                in_copy(ci + 2).start()
        out_copy(n_ch - 2).wait()
        out_copy(n_ch - 1).wait()

    return _k(dense)
